# baseline (speedup 1.0000x reference)
import jax
import jax.numpy as jnp
from jax.experimental import pallas as pl

_BLK = 4000


def _weights_body(x_ref, w1_ref, b1_ref, w2_ref, b2_ref, o_ref):
    x = x_ref[...]
    h = jnp.tanh(jnp.dot(x, w1_ref[...], preferred_element_type=jnp.float32)
                 + b1_ref[...])
    o_ref[...] = jnp.dot(h, w2_ref[...], preferred_element_type=jnp.float32) + b2_ref[...]


def kernel(proposed_samples, W1, b1, W2, b2, u):
    N, D = proposed_samples.shape
    H = W1.shape[1]
    b1r = b1.reshape(1, H)
    b2r = b2.reshape(1, 1)
    grid = N // _BLK
    a = pl.pallas_call(
        _weights_body,
        grid=(grid,),
        in_specs=[
            pl.BlockSpec((_BLK, D), lambda i: (i, 0)),
            pl.BlockSpec((D, H), lambda i: (0, 0)),
            pl.BlockSpec((1, H), lambda i: (0, 0)),
            pl.BlockSpec((H, 1), lambda i: (0, 0)),
            pl.BlockSpec((1, 1), lambda i: (0, 0)),
        ],
        out_specs=pl.BlockSpec((_BLK, 1), lambda i: (i, 0)),
        out_shape=jax.ShapeDtypeStruct((N, 1), jnp.float32),
    )(proposed_samples, W1, b1r, W2, b2r)
    w = jnp.exp(a.squeeze(-1))
    normalized = w / jnp.sum(w)
    cdf = jnp.cumsum(normalized)
    cat = jnp.searchsorted(cdf, u)
    cat = jnp.clip(cat, 0, N - 1)
    return proposed_samples[cat]


# speedup vs baseline: 4.5151x; 4.5151x over previous
import functools

import jax
import jax.numpy as jnp
from jax import lax
from jax.experimental import pallas as pl
from jax.experimental.pallas import tpu as pltpu
from jax.experimental.pallas import tpu_sc as plsc

_N = 1000000
_D = 16
_H = 64
_BLK = 4000

_NB = _N // 16
_CT_PAD = 65536
_CV = 50
_CQ = _CV * 16
_NCHUNK = _N // _CQ
_NW = 32


def _weights_body(x_ref, w1_ref, b1_ref, w2_ref, b2_ref, o_ref):
    x = x_ref[...]
    h = jnp.tanh(jnp.dot(x, w1_ref[...], preferred_element_type=jnp.float32)
                 + b1_ref[...])
    o_ref[...] = jnp.dot(h, w2_ref[...],
                         preferred_element_type=jnp.float32) + b2_ref[...]


def _sc_body(ct_hbm, cdf2_hbm, u_hbm, x_hbm, out_hbm,
             ct_v, u_v, jq_v, fine_v, p_v, rows_v, sem):
    cid = lax.axis_index("c")
    sid = lax.axis_index("s")
    wid = sid * 2 + cid
    c_lo = (_NCHUNK * wid) // _NW
    c_hi = (_NCHUNK * (wid + 1)) // _NW

    pltpu.sync_copy(ct_hbm, ct_v)
    iota = lax.iota(jnp.int32, 16)

    def chunk_body(c, carry):
        qbase = c * _CQ
        pltpu.sync_copy(u_hbm.at[pl.ds(qbase, _CQ)], u_v)

        def coarse(i, carry2):
            u16 = u_v[pl.ds(i * 16, 16)]

            def lvl_body(l, base):
                step = jnp.int32(32768) >> l
                cval = plsc.load_gather(ct_v, [base + (step - 1)])
                return base + jnp.where(cval < u16, step, 0)

            base = lax.fori_loop(0, 16, lvl_body, jnp.zeros(16, jnp.int32))
            jq_v[pl.ds(i * 16, 16)] = jnp.minimum(base, _NB - 1)
            return carry2

        lax.fori_loop(0, _CV, coarse, 0)
        pltpu.async_copy(cdf2_hbm.at[jq_v], fine_v, sem).wait()

        def fine(i, carry2):
            u16 = u_v[pl.ds(i * 16, 16)]
            jq16 = jq_v[pl.ds(i * 16, 16)]
            row = i * 16 + iota

            def k_body(k, cnt):
                col = jnp.zeros(16, jnp.int32) + k
                v = plsc.load_gather(fine_v, [row, col])
                return cnt + jnp.where(v < u16, 1, 0)

            cnt = lax.fori_loop(0, 16, k_body, jnp.zeros(16, jnp.int32))
            p_v[pl.ds(i * 16, 16)] = jnp.minimum(jq16 * 16 + cnt, _N - 1)
            return carry2

        lax.fori_loop(0, _CV, fine, 0)
        pltpu.async_copy(x_hbm.at[p_v], rows_v, sem).wait()
        pltpu.sync_copy(rows_v, out_hbm.at[pl.ds(qbase, _CQ)])
        return carry

    lax.fori_loop(c_lo, c_hi, chunk_body, 0)


_sc_sample = functools.partial(
    pl.kernel,
    mesh=plsc.VectorSubcoreMesh(core_axis_name="c", subcore_axis_name="s"),
    out_type=jax.ShapeDtypeStruct((_N, _D), jnp.float32),
    compiler_params=pltpu.CompilerParams(
        needs_layout_passes=False, use_tc_tiling_on_sc=False),
    scratch_types=[
        pltpu.VMEM((_CT_PAD,), jnp.float32),
        pltpu.VMEM((_CQ,), jnp.float32),
        pltpu.VMEM((_CQ,), jnp.int32),
        pltpu.VMEM((_CQ, 16), jnp.float32),
        pltpu.VMEM((_CQ,), jnp.int32),
        pltpu.VMEM((_CQ, _D), jnp.float32),
        pltpu.SemaphoreType.DMA,
    ],
)(_sc_body)


def kernel(proposed_samples, W1, b1, W2, b2, u):
    N, D = proposed_samples.shape
    H = W1.shape[1]
    b1r = b1.reshape(1, H)
    b2r = b2.reshape(1, 1)
    a = pl.pallas_call(
        _weights_body,
        grid=(N // _BLK,),
        in_specs=[
            pl.BlockSpec((_BLK, D), lambda i: (i, 0)),
            pl.BlockSpec((D, H), lambda i: (0, 0)),
            pl.BlockSpec((1, H), lambda i: (0, 0)),
            pl.BlockSpec((H, 1), lambda i: (0, 0)),
            pl.BlockSpec((1, 1), lambda i: (0, 0)),
        ],
        out_specs=pl.BlockSpec((_BLK, 1), lambda i: (i, 0)),
        out_shape=jax.ShapeDtypeStruct((N, 1), jnp.float32),
    )(proposed_samples, W1, b1r, W2, b2r)
    w = jnp.exp(a.squeeze(-1))
    normalized = w / jnp.sum(w)
    cdf = jnp.cumsum(normalized)
    cdf2 = cdf.reshape(_NB, 16)
    ct = jnp.concatenate(
        [cdf2[:, 15], jnp.full((_CT_PAD - _NB,), 2.0, jnp.float32)])
    return _sc_sample(ct, cdf2, u, proposed_samples)


# speedup vs baseline: 7.7765x; 1.7223x over previous
import functools

import jax
import jax.numpy as jnp
from jax import lax
from jax.experimental import pallas as pl
from jax.experimental.pallas import tpu as pltpu
from jax.experimental.pallas import tpu_sc as plsc

_N = 1000000
_D = 16
_H = 64
_BLK = 4000

_NB = _N // 16
_CT_PAD = 65536
_CV = 50
_CQ = _CV * 16
_NCHUNK = _N // _CQ
_NW = 32
_UNROLL = 5


def _weights_body(x_ref, w1_ref, b1_ref, w2_ref, b2_ref, o_ref):
    x = x_ref[...]
    h = jnp.tanh(jnp.dot(x, w1_ref[...], preferred_element_type=jnp.float32)
                 + b1_ref[...])
    o_ref[...] = jnp.dot(h, w2_ref[...],
                         preferred_element_type=jnp.float32) + b2_ref[...]


def _sc_body(ct_hbm, cdf2_hbm, u_hbm, x_hbm, out_hbm,
             ct_v, u_v, jq_v, fine_v, p_v, rows_v, sem):
    cid = lax.axis_index("c")
    sid = lax.axis_index("s")
    wid = sid * 2 + cid
    c_lo = (_NCHUNK * wid) // _NW
    c_hi = (_NCHUNK * (wid + 1)) // _NW

    pltpu.sync_copy(ct_hbm, ct_v)
    iota = lax.iota(jnp.int32, 16)

    def chunk_body(c, carry):
        qbase = c * _CQ
        pltpu.sync_copy(u_hbm.at[pl.ds(qbase, _CQ)], u_v)

        def coarse(ii, carry2):
            i0 = ii * _UNROLL
            u16s = [u_v[pl.ds((i0 + j) * 16, 16)] for j in range(_UNROLL)]

            def lvl_body(l, bases):
                step = jnp.int32(32768) >> l
                cvals = [plsc.load_gather(ct_v, [b + (step - 1)])
                         for b in bases]
                return tuple(
                    b + jnp.where(c < u, step, 0)
                    for b, c, u in zip(bases, cvals, u16s))

            zero = jnp.zeros(16, jnp.int32)
            bases = lax.fori_loop(0, 16, lvl_body, (zero,) * _UNROLL)
            for j in range(_UNROLL):
                jq_v[pl.ds((i0 + j) * 16, 16)] = jnp.minimum(bases[j], _NB - 1)
            return carry2

        lax.fori_loop(0, _CV // _UNROLL, coarse, 0)
        pltpu.async_copy(cdf2_hbm.at[jq_v], fine_v, sem).wait()

        def fine(ii, carry2):
            i0 = ii * _UNROLL
            u16s = [u_v[pl.ds((i0 + j) * 16, 16)] for j in range(_UNROLL)]
            rows = [(i0 + j) * 16 + iota for j in range(_UNROLL)]

            def k_body(k, cnts):
                col = jnp.zeros(16, jnp.int32) + k
                vals = [plsc.load_gather(fine_v, [r, col]) for r in rows]
                return tuple(
                    cn + jnp.where(v < u, 1, 0)
                    for cn, v, u in zip(cnts, vals, u16s))

            zero = jnp.zeros(16, jnp.int32)
            cnts = lax.fori_loop(0, 16, k_body, (zero,) * _UNROLL)
            for j in range(_UNROLL):
                jq16 = jq_v[pl.ds((i0 + j) * 16, 16)]
                p_v[pl.ds((i0 + j) * 16, 16)] = jnp.minimum(
                    jq16 * 16 + cnts[j], _N - 1)
            return carry2

        lax.fori_loop(0, _CV // _UNROLL, fine, 0)
        pltpu.async_copy(x_hbm.at[p_v], rows_v, sem).wait()
        pltpu.sync_copy(rows_v, out_hbm.at[pl.ds(qbase, _CQ)])
        return carry

    lax.fori_loop(c_lo, c_hi, chunk_body, 0)


_sc_sample = functools.partial(
    pl.kernel,
    mesh=plsc.VectorSubcoreMesh(core_axis_name="c", subcore_axis_name="s"),
    out_type=jax.ShapeDtypeStruct((_N, _D), jnp.float32),
    compiler_params=pltpu.CompilerParams(
        needs_layout_passes=False, use_tc_tiling_on_sc=False),
    scratch_types=[
        pltpu.VMEM((_CT_PAD,), jnp.float32),
        pltpu.VMEM((_CQ,), jnp.float32),
        pltpu.VMEM((_CQ,), jnp.int32),
        pltpu.VMEM((_CQ, 16), jnp.float32),
        pltpu.VMEM((_CQ,), jnp.int32),
        pltpu.VMEM((_CQ, _D), jnp.float32),
        pltpu.SemaphoreType.DMA,
    ],
)(_sc_body)


def kernel(proposed_samples, W1, b1, W2, b2, u):
    N, _ = proposed_samples.shape
    h = jnp.tanh(proposed_samples @ W1 + b1)
    logits = (h @ W2 + b2).squeeze(-1)
    w = jnp.exp(logits)
    normalized = w / jnp.sum(w)
    cdf = jnp.cumsum(normalized)
    cdf2 = cdf.reshape(_NB, 16)
    ct = jnp.concatenate(
        [cdf2[:, 15], jnp.full((_CT_PAD - _NB,), 2.0, jnp.float32)])
    return _sc_sample(ct, cdf2, u, proposed_samples)
